# baseline (speedup 1.0000x reference)
import functools

import jax
import jax.numpy as jnp
from jax import lax
from jax.experimental import pallas as pl
from jax.experimental.pallas import tpu as pltpu
from jax.experimental.pallas import tpu_sc as plsc

NC = 2
NS = 16
NW = NC * NS

BATCH = 16384
N_FIELDS = 26
OUT_DIM = 64
TOTAL = BATCH * N_FIELDS
CHUNK = 128
N_CHUNKS = TOTAL // CHUNK
CPW = N_CHUNKS // NW
NBUF = 4
N_GROUPS = CPW // NBUF


def _body(table_hbm, idx_hbm, out_hbm, idx_v, bufs, sems):
    wid = lax.axis_index("s") * NC + lax.axis_index("c")
    chunk0 = wid * CPW

    pltpu.sync_copy(idx_hbm.at[pl.ds(chunk0, CPW)], idx_v)

    def start_gather(j, b):
        pltpu.make_async_copy(
            table_hbm.at[idx_v.at[j]], bufs[b], sems[b]
        ).start()

    def wait_gather(j, b):
        pltpu.make_async_copy(
            table_hbm.at[idx_v.at[j]], bufs[b], sems[b]
        ).wait()

    def store(j, b):
        row0 = (chunk0 + j) * CHUNK
        pltpu.sync_copy(bufs[b], out_hbm.at[pl.ds(row0, CHUNK)])

    for b in range(NBUF):
        start_gather(b, b)

    def group(g, _):
        for b in range(NBUF):
            j = g * NBUF + b
            wait_gather(j, b)
            store(j, b)
            start_gather(j + NBUF, b)
        return 0

    lax.fori_loop(0, N_GROUPS - 1, group, 0)

    for b in range(NBUF):
        j = (N_GROUPS - 1) * NBUF + b
        wait_gather(j, b)
        store(j, b)


@jax.jit
def _embed_lookup(batch_flat, emb_weight):
    mesh = plsc.VectorSubcoreMesh(core_axis_name="c", subcore_axis_name="s")
    run = pl.kernel(
        _body,
        out_type=jax.ShapeDtypeStruct((TOTAL, OUT_DIM), jnp.float32),
        mesh=mesh,
        scratch_types=[
            pltpu.VMEM((CPW, CHUNK), jnp.int32),
            [pltpu.VMEM((CHUNK, OUT_DIM), jnp.float32) for _ in range(NBUF)],
            [pltpu.SemaphoreType.DMA for _ in range(NBUF)],
        ],
        compiler_params=pltpu.CompilerParams(use_tc_tiling_on_sc=False),
    )
    return run(emb_weight, batch_flat)


def kernel(batch, emb_weight):
    idx = batch.astype(jnp.int32).reshape(N_CHUNKS, CHUNK)
    out = _embed_lookup(idx, emb_weight)
    return out.reshape(BATCH, N_FIELDS, OUT_DIM)
